# baseline (speedup 1.0000x reference)
import functools

import jax
import jax.numpy as jnp
from jax import lax
from jax.experimental import pallas as pl
from jax.experimental.pallas import tpu as pltpu
from jax.experimental.pallas import tpu_sc as plsc

D_MODEL_K = 768
N_EXP_K = 64
TOPK_K = 2
LANES = 16


def _matmul_body(x0_ref, x1_ref, wt_ref, b_ref, g_ref):
    wt = wt_ref[...]
    b = b_ref[...]
    h = x0_ref.shape[0]
    g_ref[:h, :] = (
        jnp.dot(x0_ref[...], wt, preferred_element_type=jnp.float32) + b
    )
    g_ref[h:, :] = (
        jnp.dot(x1_ref[...], wt, preferred_element_type=jnp.float32) + b
    )


def _logits(inp, wt, b2, bt, blk0, nblk):
    h = bt // 2
    return pl.pallas_call(
        _matmul_body,
        grid=(nblk,),
        in_specs=[
            pl.BlockSpec((h, D_MODEL_K), lambda i: (2 * (blk0 + i), 0)),
            pl.BlockSpec((h, D_MODEL_K), lambda i: (2 * (blk0 + i) + 1, 0)),
            pl.BlockSpec((D_MODEL_K, N_EXP_K), lambda i: (0, 0)),
            pl.BlockSpec((1, N_EXP_K), lambda i: (0, 0)),
        ],
        out_specs=pl.BlockSpec((bt, N_EXP_K), lambda i: (i, 0)),
        out_shape=jax.ShapeDtypeStruct((nblk * bt, N_EXP_K), jnp.float32),
    )(inp, inp, wt, b2)


def _sc_topk(logits_flat, n_tok):
    nw = 32
    tpw = n_tok // nw
    mesh = plsc.VectorSubcoreMesh(core_axis_name="c", subcore_axis_name="s")

    @functools.partial(
        pl.kernel,
        out_type=(
            jax.ShapeDtypeStruct((TOPK_K * n_tok,), jnp.int32),
            jax.ShapeDtypeStruct((TOPK_K * n_tok,), jnp.float32),
        ),
        mesh=mesh,
        compiler_params=pltpu.CompilerParams(needs_layout_passes=False),
        scratch_types=[
            pltpu.VMEM((tpw * N_EXP_K,), jnp.float32),
            pltpu.VMEM((TOPK_K * tpw,), jnp.int32),
            pltpu.VMEM((TOPK_K * tpw,), jnp.float32),
        ],
    )
    def topk_kernel(logits_hbm, idx_hbm, score_hbm, buf, idxb, scoreb):
        wid = lax.axis_index("s") * 2 + lax.axis_index("c")
        t0 = wid * tpw
        pltpu.sync_copy(
            logits_hbm.at[pl.ds(t0 * N_EXP_K, tpw * N_EXP_K)], buf
        )

        def group(g, carry):
            base = g * LANES
            row = base + lax.iota(jnp.int32, LANES)
            flat = row * N_EXP_K
            m1 = plsc.load_gather(buf, [flat])
            i1 = jnp.zeros((LANES,), jnp.int32)
            m2 = jnp.full((LANES,), -jnp.inf, jnp.float32)
            i2 = jnp.zeros((LANES,), jnp.int32)
            for e in range(1, N_EXP_K):
                col = jnp.full((LANES,), e, jnp.int32)
                v = plsc.load_gather(buf, [flat + e])
                g1 = v > m1
                g2 = v > m2
                i2 = jnp.where(g1, i1, jnp.where(g2, col, i2))
                m2 = jnp.where(g1, m1, jnp.where(g2, v, m2))
                i1 = jnp.where(g1, col, i1)
                m1 = jnp.where(g1, v, m1)
            e2 = jnp.exp(m2 - m1)
            denom = 1.0 + e2
            st = TOPK_K * row
            plsc.store_scatter(idxb, [st], i1)
            plsc.store_scatter(idxb, [st + 1], i2)
            plsc.store_scatter(scoreb, [st], 1.0 / denom)
            plsc.store_scatter(scoreb, [st + 1], e2 / denom)
            return carry

        lax.fori_loop(0, tpw // LANES, group, 0)
        pltpu.sync_copy(idxb, idx_hbm.at[pl.ds(TOPK_K * t0, TOPK_K * tpw)])
        pltpu.sync_copy(scoreb, score_hbm.at[pl.ds(TOPK_K * t0, TOPK_K * tpw)])

    return topk_kernel(logits_flat)


def kernel(inp, W, b):
    n_tok = inp.shape[0]
    bt = 2048
    n_chunks = 8
    blk_per_chunk = n_tok // bt // n_chunks
    chunk = blk_per_chunk * bt
    wt = W.T
    b2 = b[None, :]
    idx_parts, score_parts = [], []
    for c in range(n_chunks):
        logits_c = _logits(inp, wt, b2, bt, c * blk_per_chunk, blk_per_chunk)
        idx_c, score_c = _sc_topk(logits_c.reshape(-1), chunk)
        idx_parts.append(idx_c)
        score_parts.append(score_c)
    idx = jnp.concatenate(idx_parts)
    score = jnp.concatenate(score_parts).reshape(n_tok, 1, TOPK_K)
    return (idx, score)


# speedup vs baseline: 1.1065x; 1.1065x over previous
import functools

import jax
import jax.numpy as jnp
from jax import lax
from jax.experimental import pallas as pl
from jax.experimental.pallas import tpu as pltpu
from jax.experimental.pallas import tpu_sc as plsc

D_MODEL_K = 768
N_EXP_K = 64
TOPK_K = 2
LANES = 16


def _matmul_body(x0_ref, x1_ref, wt_ref, b_ref, g_ref):
    wt = wt_ref[...]
    b = b_ref[...]
    h = x0_ref.shape[0]
    g_ref[:h, :] = (
        jnp.dot(x0_ref[...], wt, preferred_element_type=jnp.float32) + b
    )
    g_ref[h:, :] = (
        jnp.dot(x1_ref[...], wt, preferred_element_type=jnp.float32) + b
    )


def _logits(inp, wt, b2, bt, blk0, nblk):
    h = bt // 2
    return pl.pallas_call(
        _matmul_body,
        grid=(nblk,),
        in_specs=[
            pl.BlockSpec((h, D_MODEL_K), lambda i: (2 * (blk0 + i), 0)),
            pl.BlockSpec((h, D_MODEL_K), lambda i: (2 * (blk0 + i) + 1, 0)),
            pl.BlockSpec((D_MODEL_K, N_EXP_K), lambda i: (0, 0)),
            pl.BlockSpec((1, N_EXP_K), lambda i: (0, 0)),
        ],
        out_specs=pl.BlockSpec((bt, N_EXP_K), lambda i: (i, 0)),
        out_shape=jax.ShapeDtypeStruct((nblk * bt, N_EXP_K), jnp.float32),
    )(inp, inp, wt, b2)


def _sc_topk(logits_flat, n_tok):
    nw = 32
    tpw = n_tok // nw
    mesh = plsc.VectorSubcoreMesh(core_axis_name="c", subcore_axis_name="s")

    @functools.partial(
        pl.kernel,
        out_type=(
            jax.ShapeDtypeStruct((TOPK_K * n_tok,), jnp.int32),
            jax.ShapeDtypeStruct((TOPK_K * n_tok,), jnp.float32),
        ),
        mesh=mesh,
        compiler_params=pltpu.CompilerParams(needs_layout_passes=False),
        scratch_types=[
            pltpu.VMEM((tpw * N_EXP_K,), jnp.float32),
            pltpu.VMEM((TOPK_K * tpw,), jnp.int32),
            pltpu.VMEM((TOPK_K * tpw,), jnp.float32),
        ],
    )
    def topk_kernel(logits_hbm, idx_hbm, score_hbm, buf, idxb, scoreb):
        wid = lax.axis_index("s") * 2 + lax.axis_index("c")
        t0 = wid * tpw
        pltpu.sync_copy(
            logits_hbm.at[pl.ds(t0 * N_EXP_K, tpw * N_EXP_K)], buf
        )

        def group(g, carry):
            base = g * LANES
            row = base + lax.iota(jnp.int32, LANES)
            flat = row * N_EXP_K
            m1 = plsc.load_gather(buf, [flat])
            i1 = jnp.zeros((LANES,), jnp.int32)
            m2 = jnp.full((LANES,), -jnp.inf, jnp.float32)
            i2 = jnp.zeros((LANES,), jnp.int32)
            for e in range(1, N_EXP_K):
                col = jnp.full((LANES,), e, jnp.int32)
                v = plsc.load_gather(buf, [flat + e])
                g1 = v > m1
                g2 = v > m2
                i2 = jnp.where(g1, i1, jnp.where(g2, col, i2))
                m2 = jnp.where(g1, m1, jnp.where(g2, v, m2))
                i1 = jnp.where(g1, col, i1)
                m1 = jnp.where(g1, v, m1)
            e2 = jnp.exp(m2 - m1)
            denom = 1.0 + e2
            st = TOPK_K * row
            plsc.store_scatter(idxb, [st], i1)
            plsc.store_scatter(idxb, [st + 1], i2)
            plsc.store_scatter(scoreb, [st], 1.0 / denom)
            plsc.store_scatter(scoreb, [st + 1], e2 / denom)
            return carry

        lax.fori_loop(0, tpw // LANES, group, 0)
        pltpu.sync_copy(idxb, idx_hbm.at[pl.ds(TOPK_K * t0, TOPK_K * tpw)])
        pltpu.sync_copy(scoreb, score_hbm.at[pl.ds(TOPK_K * t0, TOPK_K * tpw)])

    return topk_kernel(logits_flat)


def kernel(inp, W, b):
    n_tok = inp.shape[0]
    bt = 4096
    n_chunks = 2
    blk_per_chunk = n_tok // bt // n_chunks
    chunk = blk_per_chunk * bt
    wt = W.T
    b2 = b[None, :]
    idx_parts, score_parts = [], []
    for c in range(n_chunks):
        logits_c = _logits(inp, wt, b2, bt, c * blk_per_chunk, blk_per_chunk)
        idx_c, score_c = _sc_topk(logits_c.reshape(-1), chunk)
        idx_parts.append(idx_c)
        score_parts.append(score_c)
    idx = jnp.concatenate(idx_parts)
    score = jnp.concatenate(score_parts).reshape(n_tok, 1, TOPK_K)
    return (idx, score)


# speedup vs baseline: 1.1065x; 1.0000x over previous
import functools

import jax
import jax.numpy as jnp
from jax import lax
from jax.experimental import pallas as pl
from jax.experimental.pallas import tpu as pltpu
from jax.experimental.pallas import tpu_sc as plsc

D_MODEL_K = 768
N_EXP_K = 64
TOPK_K = 2
LANES = 16


def _matmul_body(x_ref, wt_ref, b_ref, g_ref):
    g_ref[...] = (
        jnp.dot(x_ref[...], wt_ref[...], preferred_element_type=jnp.float32)
        + b_ref[...]
    )


def _logits(inp, wt, b2, bt, blk0, nblk):
    return pl.pallas_call(
        _matmul_body,
        grid=(nblk,),
        in_specs=[
            pl.BlockSpec((bt, D_MODEL_K), lambda i: (blk0 + i, 0)),
            pl.BlockSpec((D_MODEL_K, N_EXP_K), lambda i: (0, 0)),
            pl.BlockSpec((1, N_EXP_K), lambda i: (0, 0)),
        ],
        out_specs=pl.BlockSpec((bt, N_EXP_K), lambda i: (i, 0)),
        out_shape=jax.ShapeDtypeStruct((nblk * bt, N_EXP_K), jnp.float32),
    )(inp, wt, b2)


def _sc_topk(logits_flat, n_tok):
    nw = 32
    tpw = n_tok // nw
    mesh = plsc.VectorSubcoreMesh(core_axis_name="c", subcore_axis_name="s")

    @functools.partial(
        pl.kernel,
        out_type=(
            jax.ShapeDtypeStruct((TOPK_K * n_tok,), jnp.int32),
            jax.ShapeDtypeStruct((TOPK_K * n_tok,), jnp.float32),
        ),
        mesh=mesh,
        compiler_params=pltpu.CompilerParams(needs_layout_passes=False),
        scratch_types=[
            pltpu.VMEM((tpw * N_EXP_K,), jnp.float32),
            pltpu.VMEM((TOPK_K * tpw,), jnp.int32),
            pltpu.VMEM((TOPK_K * tpw,), jnp.float32),
        ],
    )
    def topk_kernel(logits_hbm, idx_hbm, score_hbm, buf, idxb, scoreb):
        wid = lax.axis_index("s") * 2 + lax.axis_index("c")
        t0 = wid * tpw
        pltpu.sync_copy(
            logits_hbm.at[pl.ds(t0 * N_EXP_K, tpw * N_EXP_K)], buf
        )

        def group(g, carry):
            base = g * LANES
            row = base + lax.iota(jnp.int32, LANES)
            flat = row * N_EXP_K
            m1 = plsc.load_gather(buf, [flat])
            i1 = jnp.zeros((LANES,), jnp.int32)
            m2 = jnp.full((LANES,), -jnp.inf, jnp.float32)
            i2 = jnp.zeros((LANES,), jnp.int32)
            for e in range(1, N_EXP_K):
                col = jnp.full((LANES,), e, jnp.int32)
                v = plsc.load_gather(buf, [flat + e])
                g1 = v > m1
                g2 = v > m2
                i2 = jnp.where(g1, i1, jnp.where(g2, col, i2))
                m2 = jnp.where(g1, m1, jnp.where(g2, v, m2))
                i1 = jnp.where(g1, col, i1)
                m1 = jnp.where(g1, v, m1)
            e2 = jnp.exp(m2 - m1)
            denom = 1.0 + e2
            st = TOPK_K * row
            plsc.store_scatter(idxb, [st], i1)
            plsc.store_scatter(idxb, [st + 1], i2)
            plsc.store_scatter(scoreb, [st], 1.0 / denom)
            plsc.store_scatter(scoreb, [st + 1], e2 / denom)
            return carry

        lax.fori_loop(0, tpw // LANES, group, 0)
        pltpu.sync_copy(idxb, idx_hbm.at[pl.ds(TOPK_K * t0, TOPK_K * tpw)])
        pltpu.sync_copy(scoreb, score_hbm.at[pl.ds(TOPK_K * t0, TOPK_K * tpw)])

    return topk_kernel(logits_flat)


def kernel(inp, W, b):
    n_tok = inp.shape[0]
    bt = 4096
    n_chunks = 2
    blk_per_chunk = n_tok // bt // n_chunks
    chunk = blk_per_chunk * bt
    wt = W.T
    b2 = b[None, :]
    idx_parts, score_parts = [], []
    for c in range(n_chunks):
        logits_c = _logits(inp, wt, b2, bt, c * blk_per_chunk, blk_per_chunk)
        idx_c, score_c = _sc_topk(logits_c.reshape(-1), chunk)
        idx_parts.append(idx_c)
        score_parts.append(score_c)
    idx = jnp.concatenate(idx_parts)
    score = jnp.concatenate(score_parts).reshape(n_tok, 1, TOPK_K)
    return (idx, score)
